# baseline (speedup 1.0000x reference)
import functools
import jax
import jax.numpy as jnp
from jax import lax
from jax.experimental import pallas as pl
from jax.experimental.pallas import tpu as pltpu
from jax.experimental.pallas import tpu_sc as plsc

BATCH = 4096
SEQ = 50
DIM = 128
NC, NS = 2, 16
NW = NC * NS
ROWS_PER_W = BATCH // NW
NBUF = 5


@functools.partial(
    pl.kernel,
    out_type=jax.ShapeDtypeStruct((SEQ, BATCH, DIM), jnp.float32),
    mesh=plsc.VectorSubcoreMesh(core_axis_name="c", subcore_axis_name="s"),
    compiler_params=pltpu.CompilerParams(
        disable_bounds_checks=True,
        disable_semaphore_checks=True,
    ),
    scratch_types=(
        [pltpu.VMEM((SEQ, ROWS_PER_W), jnp.int32)]
        + [pltpu.VMEM((ROWS_PER_W, DIM), jnp.float32) for _ in range(NBUF)]
        + [pltpu.SemaphoreType.DMA for _ in range(2 * NBUF)]
    ),
)
def _gather_kernel(xt_hbm, table_hbm, out_hbm, idx_v, *scratch):
    bufs = scratch[:NBUF]
    gsem = scratch[NBUF:2 * NBUF]
    ssem = scratch[2 * NBUF:]
    wid = lax.axis_index("s") * NC + lax.axis_index("c")
    base = wid * ROWS_PER_W
    pltpu.sync_copy(xt_hbm.at[pl.ds(0, 8), pl.ds(base, ROWS_PER_W)],
                    idx_v.at[pl.ds(0, 8)])

    def gather_start(b, s):
        pltpu.async_copy(table_hbm.at[idx_v.at[s]], bufs[b], gsem[b])

    def gather_wait(b, s):
        pltpu.make_async_copy(table_hbm.at[idx_v.at[s]], bufs[b],
                              gsem[b]).wait()

    def store_start(b, s):
        pltpu.async_copy(bufs[b], out_hbm.at[s].at[pl.ds(base, ROWS_PER_W)],
                         ssem[b])

    def store_wait(b, s):
        pltpu.make_async_copy(bufs[b],
                              out_hbm.at[s].at[pl.ds(base, ROWS_PER_W)],
                              ssem[b]).wait()

    for b in range(NBUF):
        gather_start(b, b)
    pltpu.sync_copy(xt_hbm.at[pl.ds(8, SEQ - 8), pl.ds(base, ROWS_PER_W)],
                    idx_v.at[pl.ds(8, SEQ - 8)])

    def body(t, carry):
        for b in range(NBUF):
            s = t * NBUF + b
            gather_wait(b, s)
            store_start(b, s)
        for b in range(NBUF):
            s = t * NBUF + b
            sn = s + NBUF

            @pl.when(sn < SEQ)
            def _():
                store_wait(b, s)
                gather_start(b, sn)

        return carry

    lax.fori_loop(0, SEQ // NBUF, body, 0)
    for b in range(NBUF):
        store_wait(b, SEQ - NBUF + b)


def kernel(x, table):
    xt = jnp.transpose(x.astype(jnp.int32))
    out_t = _gather_kernel(xt, table)
    return jnp.transpose(out_t, (1, 0, 2))


# speedup vs baseline: 1.0297x; 1.0297x over previous
import functools
import jax
import jax.numpy as jnp
from jax import lax
from jax.experimental import pallas as pl
from jax.experimental.pallas import tpu as pltpu
from jax.experimental.pallas import tpu_sc as plsc

BATCH = 4096
SEQ = 50
DIM = 128
NC, NS = 2, 16
NW = NC * NS
ROWS_PER_W = BATCH // NW
HALF = 2
CROWS = ROWS_PER_W // HALF
NCHUNK = SEQ * HALF
NBUF = 10


@functools.partial(
    pl.kernel,
    out_type=jax.ShapeDtypeStruct((SEQ, BATCH, DIM), jnp.float32),
    mesh=plsc.VectorSubcoreMesh(core_axis_name="c", subcore_axis_name="s"),
    compiler_params=pltpu.CompilerParams(
        disable_bounds_checks=True,
        disable_semaphore_checks=True,
    ),
    scratch_types=(
        [pltpu.VMEM((SEQ, ROWS_PER_W), jnp.int32)]
        + [pltpu.VMEM((CROWS, DIM), jnp.float32) for _ in range(NBUF)]
        + [pltpu.SemaphoreType.DMA for _ in range(2 * NBUF)]
    ),
)
def _gather_kernel(xt_hbm, table_hbm, out_hbm, idx_v, *scratch):
    bufs = scratch[:NBUF]
    gsem = scratch[NBUF:2 * NBUF]
    ssem = scratch[2 * NBUF:]
    wid = lax.axis_index("s") * NC + lax.axis_index("c")
    base = wid * ROWS_PER_W
    pltpu.sync_copy(xt_hbm.at[:, pl.ds(base, ROWS_PER_W)], idx_v)

    def gather_start(b, c):
        s, h = c // HALF, c % HALF
        pltpu.async_copy(table_hbm.at[idx_v.at[s].at[pl.ds(h * CROWS, CROWS)]],
                         bufs[b], gsem[b])

    def gather_wait(b, c):
        s, h = c // HALF, c % HALF
        pltpu.make_async_copy(
            table_hbm.at[idx_v.at[s].at[pl.ds(h * CROWS, CROWS)]],
            bufs[b], gsem[b]).wait()

    def store_start(b, c):
        s, h = c // HALF, c % HALF
        pltpu.async_copy(bufs[b],
                         out_hbm.at[s].at[pl.ds(base + h * CROWS, CROWS)],
                         ssem[b])

    def store_wait(b, c):
        s, h = c // HALF, c % HALF
        pltpu.make_async_copy(bufs[b],
                              out_hbm.at[s].at[pl.ds(base + h * CROWS, CROWS)],
                              ssem[b]).wait()

    for b in range(NBUF):
        gather_start(b, b)

    def body(t, carry):
        for b in range(NBUF):
            c = t * NBUF + b
            gather_wait(b, c)
            store_start(b, c)
        for b in range(NBUF):
            c = t * NBUF + b
            cn = c + NBUF

            @pl.when(cn < NCHUNK)
            def _():
                store_wait(b, c)
                gather_start(b, cn)

        return carry

    lax.fori_loop(0, NCHUNK // NBUF, body, 0)
    for b in range(NBUF):
        store_wait(b, NCHUNK - NBUF + b)


def kernel(x, table):
    xt = jnp.transpose(x.astype(jnp.int32))
    out_t = _gather_kernel(xt, table)
    return jnp.transpose(out_t, (1, 0, 2))
